# Initial kernel scaffold; baseline (speedup 1.0000x reference)
import jax
import jax.numpy as jnp
from jax.experimental import pallas as pl


def kernel(x, edge_index, batch, W1, b1, Wlin, blin, W2, b2, Wr):
    raise NotImplementedError("write your pallas kernel here")


# speedup vs baseline: 8.0306x; 8.0306x over previous
import functools

import jax
import jax.numpy as jnp
from jax import lax
from jax.experimental import pallas as pl
from jax.experimental.pallas import tpu as pltpu
from jax.experimental.pallas import tpu_sc as plsc

_HI = lax.Precision.HIGHEST
_NS = 16
_K = 80
_DH = 128
_NG = 64


def _dinv(d0_ref, d1_ref):
    return lax.rsqrt(d0_ref[:, 0] + d1_ref[:, 0] + 1.0)


def _sc_degree(dst4, ones_h, zeros_h, n_nodes):
    chd = dst4.shape[2]
    rpt = n_nodes // _NS
    mesh = plsc.VectorSubcoreMesh(core_axis_name="c", subcore_axis_name="s")

    @functools.partial(
        pl.kernel,
        out_type=jax.ShapeDtypeStruct((2 * n_nodes, _DH), jnp.float32),
        mesh=mesh,
        scratch_types=[
            pltpu.VMEM((_K,), jnp.int32),
            pltpu.VMEM((_K, _DH), jnp.float32),
            pltpu.VMEM_SHARED((n_nodes, _DH), jnp.float32),
        ],
    )
    def deg_kernel(dst_h, ones_hbm, zeros_hbm, out, dib, ones_v, acc):
        c = lax.axis_index("c")
        s = lax.axis_index("s")
        pltpu.sync_copy(zeros_hbm, acc.at[pl.ds(s * rpt, rpt)])
        pltpu.sync_copy(ones_hbm, ones_v)
        plsc.subcore_barrier()

        def body(j, carry):
            pltpu.sync_copy(dst_h.at[c, s, j], dib)
            pltpu.sync_copy(ones_v, acc.at[dib], add=True)
            return carry

        lax.fori_loop(0, chd, body, 0)
        plsc.subcore_barrier()
        pltpu.sync_copy(acc.at[pl.ds(s * rpt, rpt)],
                        out.at[pl.ds(c * n_nodes + s * rpt, rpt)])

    return deg_kernel(dst4, ones_h, zeros_h)


def _sc_scatter(y_flat, src3, dst3, zeros_h, n_nodes):
    ch = src3.shape[1]
    rpt = n_nodes // _NS
    mesh = plsc.VectorSubcoreMesh(core_axis_name="c", subcore_axis_name="s")

    @functools.partial(
        pl.kernel,
        out_type=jax.ShapeDtypeStruct((2 * n_nodes, _DH), jnp.float32),
        mesh=mesh,
        scratch_types=[
            pltpu.VMEM((_K,), jnp.int32),
            pltpu.VMEM((_K,), jnp.int32),
            pltpu.VMEM((_K, _DH), jnp.float32),
            pltpu.VMEM_SHARED((n_nodes, _DH), jnp.float32),
            pltpu.SemaphoreType.DMA,
        ],
    )
    def conv_kernel(y_h, src_h, dst_h, zeros_hbm, out, sib, dib, r0, acc,
                    sem):
        c = lax.axis_index("c")
        s = lax.axis_index("s")
        pltpu.sync_copy(zeros_hbm, acc.at[pl.ds(s * rpt, rpt)])
        plsc.subcore_barrier()
        base = c * n_nodes

        def body(j, carry):
            pltpu.sync_copy(src_h.at[s, j], sib)

            def fix(q, carry2):
                sib[pl.ds(q * 16, 16)] = sib[pl.ds(q * 16, 16)] + base
                return carry2

            lax.fori_loop(0, _K // 16, fix, 0)
            pltpu.async_copy(y_h.at[sib], r0, sem).wait()
            pltpu.sync_copy(dst_h.at[s, j], dib)
            pltpu.sync_copy(r0, acc.at[dib], add=True)
            return carry

        lax.fori_loop(0, ch, body, 0)
        plsc.subcore_barrier()
        pltpu.sync_copy(acc.at[pl.ds(s * rpt, rpt)],
                        out.at[pl.ds(c * n_nodes + s * rpt, rpt)])

    return conv_kernel(y_flat, src3, dst3, zeros_h)


def _tc_scale_matmul(x, w1, d0, d1, blk):
    n, din = x.shape
    nh = w1.shape[1]
    g = n // blk

    def body(x_ref, w_ref, d0_ref, d1_ref, y_ref):
        dinv = _dinv(d0_ref, d1_ref)
        xw = jnp.dot(x_ref[...], w_ref[...], precision=_HI,
                     preferred_element_type=jnp.float32)
        y = dinv[:, None] * xw
        y_ref[0] = y[:, :_DH]
        y_ref[1] = y[:, _DH:]

    return pl.pallas_call(
        body,
        grid=(g,),
        in_specs=[
            pl.BlockSpec((blk, din), lambda i: (i, 0)),
            pl.BlockSpec((din, nh), lambda i: (0, 0)),
            pl.BlockSpec((blk, 16), lambda i: (i, 0)),
            pl.BlockSpec((blk, 16), lambda i: (i, 0)),
        ],
        out_specs=pl.BlockSpec((2, blk, _DH), lambda i: (0, i, 0)),
        out_shape=jax.ShapeDtypeStruct((2, n, _DH), jnp.float32),
    )(x, w1, d0, d1)


def _tc_mid(s1a, s1b, y1, d0, d1, b1, wlin_t, blin, w2, blk):
    n = s1a.shape[0]
    nh = wlin_t.shape[0]
    g = n // blk

    def body(sa_ref, sb_ref, y_ref, d0_ref, d1_ref, b1_ref, wl_ref, bl_ref,
             w2_ref, out_ref):
        dinv = _dinv(d0_ref, d1_ref)
        s1 = jnp.concatenate([sa_ref[...], sb_ref[...]], axis=1)
        y1 = jnp.concatenate([y_ref[0], y_ref[1]], axis=1)
        h1 = jax.nn.relu(dinv[:, None] * (s1 + y1) + b1_ref[0])
        h = jax.nn.relu(jnp.dot(h1, wl_ref[...], precision=_HI,
                                preferred_element_type=jnp.float32) + bl_ref[0])
        y2 = dinv[:, None] * jnp.dot(h, w2_ref[...], precision=_HI,
                                     preferred_element_type=jnp.float32)
        out_ref[0] = y2[:, :_DH]
        out_ref[1] = y2[:, _DH:]

    return pl.pallas_call(
        body,
        grid=(g,),
        in_specs=[
            pl.BlockSpec((blk, _DH), lambda i: (i, 0)),
            pl.BlockSpec((blk, _DH), lambda i: (i, 0)),
            pl.BlockSpec((2, blk, _DH), lambda i: (0, i, 0)),
            pl.BlockSpec((blk, 16), lambda i: (i, 0)),
            pl.BlockSpec((blk, 16), lambda i: (i, 0)),
            pl.BlockSpec((1, nh), lambda i: (0, 0)),
            pl.BlockSpec((nh, nh), lambda i: (0, 0)),
            pl.BlockSpec((1, nh), lambda i: (0, 0)),
            pl.BlockSpec((nh, nh), lambda i: (0, 0)),
        ],
        out_specs=pl.BlockSpec((2, blk, _DH), lambda i: (0, i, 0)),
        out_shape=jax.ShapeDtypeStruct((2, n, _DH), jnp.float32),
    )(s1a, s1b, y1, d0, d1, b1, wlin_t, blin, w2)


def _tc_readout(s2a, s2b, y2, d0, d1, b2, batch3, wr, blk):
    n = s2a.shape[0]
    nh = wr.shape[0]
    g = n // blk

    def body(sa_ref, sb_ref, y_ref, d0_ref, d1_ref, b2_ref, bat_ref, wr_ref,
             out_ref, sums, counts, tg):
        p = pl.program_id(0)
        i = pl.program_id(1)
        dinv = _dinv(d0_ref, d1_ref)
        s2 = jnp.concatenate([sa_ref[...], sb_ref[...]], axis=1)
        y2 = jnp.concatenate([y_ref[0], y_ref[1]], axis=1)
        h2 = dinv[:, None] * (s2 + y2) + b2_ref[0]
        bat = bat_ref[0, 0]
        onehot = (bat[:, None] == lax.broadcasted_iota(jnp.int32, (1, _NG), 1)
                  ).astype(jnp.float32)

        @pl.when((p == 0) & (i == 0))
        def _():
            sums[...] = jnp.zeros_like(sums)
            counts[...] = jnp.zeros_like(counts)

        @pl.when(p == 0)
        def _():
            sums[...] += lax.dot_general(
                onehot, h2, (((0,), (0,)), ((), ())), precision=_HI,
                preferred_element_type=jnp.float32)
            counts[...] += jnp.sum(onehot, axis=0, keepdims=True)

        @pl.when((p == 1) & (i == 0))
        def _():
            cnt = jnp.reshape(jnp.clip(counts[...], 1.0), (_NG, 1))
            mean = sums[...] / cnt
            tg[...] = jnp.tanh(jnp.dot(mean, wr_ref[...], precision=_HI,
                                       preferred_element_type=jnp.float32))
            out_ref[...] = jnp.zeros_like(out_ref)

        @pl.when(p == 1)
        def _():
            tgb = jnp.dot(onehot, tg[...], precision=_HI,
                          preferred_element_type=jnp.float32)
            coef = jax.nn.sigmoid(jnp.sum(h2 * tgb, axis=1))
            out_ref[...] += lax.dot_general(
                onehot, coef[:, None] * h2, (((0,), (0,)), ((), ())),
                precision=_HI, preferred_element_type=jnp.float32)

    return pl.pallas_call(
        body,
        grid=(2, g),
        in_specs=[
            pl.BlockSpec((blk, _DH), lambda p, i: (i, 0)),
            pl.BlockSpec((blk, _DH), lambda p, i: (i, 0)),
            pl.BlockSpec((2, blk, _DH), lambda p, i: (0, i, 0)),
            pl.BlockSpec((blk, 16), lambda p, i: (i, 0)),
            pl.BlockSpec((blk, 16), lambda p, i: (i, 0)),
            pl.BlockSpec((1, nh), lambda p, i: (0, 0)),
            pl.BlockSpec((1, 1, blk), lambda p, i: (i, 0, 0)),
            pl.BlockSpec((nh, nh), lambda p, i: (0, 0)),
        ],
        out_specs=pl.BlockSpec((_NG, nh), lambda p, i: (0, 0)),
        out_shape=jax.ShapeDtypeStruct((_NG, nh), jnp.float32),
        scratch_shapes=[
            pltpu.VMEM((_NG, nh), jnp.float32),
            pltpu.VMEM((1, _NG), jnp.float32),
            pltpu.VMEM((_NG, nh), jnp.float32),
        ],
    )(s2a, s2b, y2, d0, d1, b2, batch3, wr)


def kernel(x, edge_index, batch, W1, b1, Wlin, blin, W2, b2, Wr):
    n, _ = x.shape
    e = edge_index.shape[1]
    npad = _NS * 8 * ((n + _NS * 8 - 1) // (_NS * 8))
    blk = npad // _NS
    ch = e // (_NS * _K)
    chd = e // (2 * _NS * _K)

    x = jnp.pad(x, ((0, npad - n), (0, 0)))
    batch = jnp.pad(batch, (0, npad - n), constant_values=_NG)

    src3 = edge_index[0].reshape(_NS, ch, _K)
    dst3 = edge_index[1].reshape(_NS, ch, _K)
    dst4 = edge_index[1].reshape(2, _NS, chd, _K)
    ones_d = jnp.ones((_K, _DH), jnp.float32)
    zeros_d = jnp.zeros((blk, _DH), jnp.float32)
    batch3 = batch.reshape(_NS, 1, blk)

    deg_flat = _sc_degree(dst4, ones_d, zeros_d, npad)
    d0 = deg_flat[:npad, :16]
    d1 = deg_flat[npad:, :16]
    y1 = _tc_scale_matmul(x, W1, d0, d1, blk)
    s1f = _sc_scatter(y1.reshape(2 * npad, _DH), src3, dst3, zeros_d, npad)
    y2 = _tc_mid(s1f[:npad], s1f[npad:], y1, d0, d1, b1.reshape(1, -1),
                 Wlin.T, blin.reshape(1, -1), W2, blk)
    s2f = _sc_scatter(y2.reshape(2 * npad, _DH), src3, dst3, zeros_d, npad)
    return _tc_readout(s2f[:npad], s2f[npad:], y2, d0, d1, b2.reshape(1, -1),
                       batch3, Wr, blk)


# speedup vs baseline: 11.6202x; 1.4470x over previous
import functools

import jax
import jax.numpy as jnp
from jax import lax
from jax.experimental import pallas as pl
from jax.experimental.pallas import tpu as pltpu
from jax.experimental.pallas import tpu_sc as plsc

_HI = lax.Precision.HIGHEST
_NS = 16
_K = 80
_DH = 128
_NG = 64


def _dinv(d0_ref, d1_ref):
    return lax.rsqrt(d0_ref[:, 0] + d1_ref[:, 0] + 1.0)


def _sc_degree(dst4, ones_h, zeros_h, n_nodes):
    chd = dst4.shape[2]
    rpt = n_nodes // _NS
    mesh = plsc.VectorSubcoreMesh(core_axis_name="c", subcore_axis_name="s")

    @functools.partial(
        pl.kernel,
        out_type=jax.ShapeDtypeStruct((2 * n_nodes, _DH), jnp.float32),
        mesh=mesh,
        scratch_types=[
            pltpu.VMEM((_K,), jnp.int32),
            pltpu.VMEM((_K,), jnp.int32),
            pltpu.VMEM((_K,), jnp.int32),
            pltpu.VMEM((_K,), jnp.int32),
            pltpu.VMEM((_K, _DH), jnp.float32),
            pltpu.VMEM_SHARED((n_nodes + 8, _DH), jnp.float32),
            pltpu.SemaphoreType.DMA,
        ],
    )
    def deg_kernel(dst_h, ones_hbm, zeros_hbm, out, d0, d1, d2, d3, ones_v,
                   acc, sem):
        c = lax.axis_index("c")
        s = lax.axis_index("s")
        pltpu.sync_copy(zeros_hbm, acc.at[pl.ds(s * rpt, rpt)])
        pltpu.sync_copy(ones_hbm, ones_v)
        plsc.subcore_barrier()
        dibs = (d0, d1, d2, d3)

        def body(j, carry):
            cps = []
            for b, dib in enumerate(dibs):
                pltpu.sync_copy(dst_h.at[c, s, j * 4 + b], dib)
                cps.append(pltpu.async_copy(ones_v, acc.at[dib], sem,
                                            add=True))
            for cp in cps:
                cp.wait()
            return carry

        lax.fori_loop(0, chd // 4, body, 0)
        plsc.subcore_barrier()
        pltpu.sync_copy(acc.at[pl.ds(s * rpt, rpt)],
                        out.at[pl.ds(c * n_nodes + s * rpt, rpt)])

    return deg_kernel(dst4, ones_h, zeros_h)


def _sc_scatter(y_flat, src3, dst3, zeros_h, n_nodes):
    ch = src3.shape[1] - 1
    rpt = n_nodes // _NS
    mesh = plsc.VectorSubcoreMesh(core_axis_name="c", subcore_axis_name="s")

    @functools.partial(
        pl.kernel,
        out_type=jax.ShapeDtypeStruct((2 * n_nodes, _DH), jnp.float32),
        mesh=mesh,
        scratch_types=[
            pltpu.VMEM((_K,), jnp.int32),
            pltpu.VMEM((_K,), jnp.int32),
            pltpu.VMEM((_K,), jnp.int32),
            pltpu.VMEM((_K,), jnp.int32),
            pltpu.VMEM((_K, _DH), jnp.float32),
            pltpu.VMEM((_K, _DH), jnp.float32),
            pltpu.VMEM_SHARED((n_nodes, _DH), jnp.float32),
            pltpu.SemaphoreType.DMA,
            pltpu.SemaphoreType.DMA,
        ],
    )
    def conv_kernel(y_h, src_h, dst_h, zeros_hbm, out, sib0, sib1, dib0,
                    dib1, r0, r1, acc, sem0, sem1):
        c = lax.axis_index("c")
        s = lax.axis_index("s")
        pltpu.sync_copy(zeros_hbm, acc.at[pl.ds(s * rpt, rpt)])
        plsc.subcore_barrier()
        base = c * n_nodes

        def load_idx(j, buf):
            pltpu.sync_copy(src_h.at[s, j], buf)

            def fix(q, carry2):
                buf[pl.ds(q * 16, 16)] = buf[pl.ds(q * 16, 16)] + base
                return carry2

            lax.fori_loop(0, _K // 16, fix, 0)

        load_idx(0, sib0)
        pltpu.async_copy(y_h.at[sib0], r0, sem0)

        def body(jj, carry):
            j0 = jj * 2
            pltpu.sync_copy(dst_h.at[s, j0], dib0)
            load_idx(j0 + 1, sib1)
            pltpu.make_async_copy(y_h.at[sib0], r0, sem0).wait()
            pltpu.async_copy(y_h.at[sib1], r1, sem1)
            load_idx(j0 + 2, sib0)
            pltpu.sync_copy(r0, acc.at[dib0], add=True)
            pltpu.async_copy(y_h.at[sib0], r0, sem0)
            pltpu.sync_copy(dst_h.at[s, j0 + 1], dib1)
            pltpu.make_async_copy(y_h.at[sib1], r1, sem1).wait()
            pltpu.sync_copy(r1, acc.at[dib1], add=True)
            return carry

        lax.fori_loop(0, ch // 2, body, 0)
        pltpu.make_async_copy(y_h.at[sib0], r0, sem0).wait()
        plsc.subcore_barrier()
        pltpu.sync_copy(acc.at[pl.ds(s * rpt, rpt)],
                        out.at[pl.ds(c * n_nodes + s * rpt, rpt)])

    return conv_kernel(y_flat, src3, dst3, zeros_h)


def _tc_scale_matmul(x, w1, d0, d1, blk):
    n, din = x.shape
    nh = w1.shape[1]
    g = n // blk

    def body(x_ref, w_ref, d0_ref, d1_ref, y_ref):
        dinv = _dinv(d0_ref, d1_ref)
        xw = jnp.dot(x_ref[...], w_ref[...], precision=_HI,
                     preferred_element_type=jnp.float32)
        y = dinv[:, None] * xw
        y_ref[0] = y[:, :_DH]
        y_ref[1] = y[:, _DH:]

    return pl.pallas_call(
        body,
        grid=(g,),
        in_specs=[
            pl.BlockSpec((blk, din), lambda i: (i, 0)),
            pl.BlockSpec((din, nh), lambda i: (0, 0)),
            pl.BlockSpec((blk, 16), lambda i: (i, 0)),
            pl.BlockSpec((blk, 16), lambda i: (i, 0)),
        ],
        out_specs=pl.BlockSpec((2, blk, _DH), lambda i: (0, i, 0)),
        out_shape=jax.ShapeDtypeStruct((2, n, _DH), jnp.float32),
    )(x, w1, d0, d1)


def _tc_mid(s1a, s1b, y1, d0, d1, b1, wlin_t, blin, w2, blk):
    n = s1a.shape[0]
    nh = wlin_t.shape[0]
    g = n // blk

    def body(sa_ref, sb_ref, y_ref, d0_ref, d1_ref, b1_ref, wl_ref, bl_ref,
             w2_ref, out_ref):
        dinv = _dinv(d0_ref, d1_ref)
        s1 = jnp.concatenate([sa_ref[...], sb_ref[...]], axis=1)
        y1 = jnp.concatenate([y_ref[0], y_ref[1]], axis=1)
        h1 = jax.nn.relu(dinv[:, None] * (s1 + y1) + b1_ref[0])
        h = jax.nn.relu(jnp.dot(h1, wl_ref[...], precision=_HI,
                                preferred_element_type=jnp.float32) + bl_ref[0])
        y2 = dinv[:, None] * jnp.dot(h, w2_ref[...], precision=_HI,
                                     preferred_element_type=jnp.float32)
        out_ref[0] = y2[:, :_DH]
        out_ref[1] = y2[:, _DH:]

    return pl.pallas_call(
        body,
        grid=(g,),
        in_specs=[
            pl.BlockSpec((blk, _DH), lambda i: (i, 0)),
            pl.BlockSpec((blk, _DH), lambda i: (i, 0)),
            pl.BlockSpec((2, blk, _DH), lambda i: (0, i, 0)),
            pl.BlockSpec((blk, 16), lambda i: (i, 0)),
            pl.BlockSpec((blk, 16), lambda i: (i, 0)),
            pl.BlockSpec((1, nh), lambda i: (0, 0)),
            pl.BlockSpec((nh, nh), lambda i: (0, 0)),
            pl.BlockSpec((1, nh), lambda i: (0, 0)),
            pl.BlockSpec((nh, nh), lambda i: (0, 0)),
        ],
        out_specs=pl.BlockSpec((2, blk, _DH), lambda i: (0, i, 0)),
        out_shape=jax.ShapeDtypeStruct((2, n, _DH), jnp.float32),
    )(s1a, s1b, y1, d0, d1, b1, wlin_t, blin, w2)


def _tc_readout(s2a, s2b, y2, d0, d1, b2, batch3, wr, blk):
    n = s2a.shape[0]
    nh = wr.shape[0]
    g = n // blk

    def body(sa_ref, sb_ref, y_ref, d0_ref, d1_ref, b2_ref, bat_ref, wr_ref,
             out_ref, sums, counts, tg):
        p = pl.program_id(0)
        i = pl.program_id(1)
        dinv = _dinv(d0_ref, d1_ref)
        s2 = jnp.concatenate([sa_ref[...], sb_ref[...]], axis=1)
        y2 = jnp.concatenate([y_ref[0], y_ref[1]], axis=1)
        h2 = dinv[:, None] * (s2 + y2) + b2_ref[0]
        bat = bat_ref[0, 0]
        onehot = (bat[:, None] == lax.broadcasted_iota(jnp.int32, (1, _NG), 1)
                  ).astype(jnp.float32)

        @pl.when((p == 0) & (i == 0))
        def _():
            sums[...] = jnp.zeros_like(sums)
            counts[...] = jnp.zeros_like(counts)

        @pl.when(p == 0)
        def _():
            sums[...] += lax.dot_general(
                onehot, h2, (((0,), (0,)), ((), ())), precision=_HI,
                preferred_element_type=jnp.float32)
            counts[...] += jnp.sum(onehot, axis=0, keepdims=True)

        @pl.when((p == 1) & (i == 0))
        def _():
            cnt = jnp.reshape(jnp.clip(counts[...], 1.0), (_NG, 1))
            mean = sums[...] / cnt
            tg[...] = jnp.tanh(jnp.dot(mean, wr_ref[...], precision=_HI,
                                       preferred_element_type=jnp.float32))
            out_ref[...] = jnp.zeros_like(out_ref)

        @pl.when(p == 1)
        def _():
            tgb = jnp.dot(onehot, tg[...], precision=_HI,
                          preferred_element_type=jnp.float32)
            coef = jax.nn.sigmoid(jnp.sum(h2 * tgb, axis=1))
            out_ref[...] += lax.dot_general(
                onehot, coef[:, None] * h2, (((0,), (0,)), ((), ())),
                precision=_HI, preferred_element_type=jnp.float32)

    return pl.pallas_call(
        body,
        grid=(2, g),
        in_specs=[
            pl.BlockSpec((blk, _DH), lambda p, i: (i, 0)),
            pl.BlockSpec((blk, _DH), lambda p, i: (i, 0)),
            pl.BlockSpec((2, blk, _DH), lambda p, i: (0, i, 0)),
            pl.BlockSpec((blk, 16), lambda p, i: (i, 0)),
            pl.BlockSpec((blk, 16), lambda p, i: (i, 0)),
            pl.BlockSpec((1, nh), lambda p, i: (0, 0)),
            pl.BlockSpec((1, 1, blk), lambda p, i: (i, 0, 0)),
            pl.BlockSpec((nh, nh), lambda p, i: (0, 0)),
        ],
        out_specs=pl.BlockSpec((_NG, nh), lambda p, i: (0, 0)),
        out_shape=jax.ShapeDtypeStruct((_NG, nh), jnp.float32),
        scratch_shapes=[
            pltpu.VMEM((_NG, nh), jnp.float32),
            pltpu.VMEM((1, _NG), jnp.float32),
            pltpu.VMEM((_NG, nh), jnp.float32),
        ],
    )(s2a, s2b, y2, d0, d1, b2, batch3, wr)


def kernel(x, edge_index, batch, W1, b1, Wlin, blin, W2, b2, Wr):
    n, _ = x.shape
    e = edge_index.shape[1]
    npad = _NS * 8 * ((n + _NS * 8 - 1) // (_NS * 8))
    blk = npad // _NS
    ch = e // (_NS * _K)
    chd = e // (2 * _NS * _K)

    x = jnp.pad(x, ((0, npad - n), (0, 0)))
    batch = jnp.pad(batch, (0, npad - n), constant_values=_NG)

    src3 = jnp.pad(edge_index[0].reshape(_NS, ch, _K), ((0, 0), (0, 1),
                                                        (0, 0)))
    dst3 = jnp.pad(edge_index[1].reshape(_NS, ch, _K), ((0, 0), (0, 1),
                                                        (0, 0)))
    chd_pad = 4 * ((chd + 3) // 4)
    dst4 = jnp.pad(edge_index[1].reshape(2, _NS, chd, _K),
                   ((0, 0), (0, 0), (0, chd_pad - chd), (0, 0)),
                   constant_values=npad)
    ones_d = jnp.ones((_K, _DH), jnp.float32)
    zeros_d = jnp.zeros((blk, _DH), jnp.float32)
    batch3 = batch.reshape(_NS, 1, blk)

    deg_flat = _sc_degree(dst4, ones_d, zeros_d, npad)
    d0 = deg_flat[:npad, :16]
    d1 = deg_flat[npad:, :16]
    y1 = _tc_scale_matmul(x, W1, d0, d1, blk)
    s1f = _sc_scatter(y1.reshape(2 * npad, _DH), src3, dst3, zeros_d, npad)
    y2 = _tc_mid(s1f[:npad], s1f[npad:], y1, d0, d1, b1.reshape(1, -1),
                 Wlin.T, blin.reshape(1, -1), W2, blk)
    s2f = _sc_scatter(y2.reshape(2 * npad, _DH), src3, dst3, zeros_d, npad)
    return _tc_readout(s2f[:npad], s2f[npad:], y2, d0, d1, b2.reshape(1, -1),
                       batch3, Wr, blk)
